# baseline (speedup 1.0000x reference)
import functools
import jax
import jax.numpy as jnp
from jax import lax
from jax.experimental import pallas as pl
from jax.experimental.pallas import tpu as pltpu, tpu_sc as plsc

B, T, D = 4, 8192, 1024
NW = 32
T_PER_W = T // NW
CHUNK_T = 16
N_CHUNK = T_PER_W // CHUNK_T
CHUNK_E = CHUNK_T * D


def _sc_body(x_hbm, emb_hbm, out_hbm, ebuf, xbuf, sem):
    cid = lax.axis_index("c")
    sid = lax.axis_index("s")
    wid = sid * 2 + cid
    t_base = wid * T_PER_W

    def chunk_body(c, _):
        e_off = (t_base + c * CHUNK_T) * D
        pltpu.sync_copy(emb_hbm.at[pl.ds(e_off, CHUNK_E)], ebuf)

        def batch_body(b, _):
            x_off = b * (T * D) + e_off
            pltpu.sync_copy(x_hbm.at[pl.ds(x_off, CHUNK_E)], xbuf)

            @plsc.parallel_loop(0, CHUNK_E, 16, unroll=8)
            def _add(o):
                xbuf[pl.ds(o, 16)] = xbuf[pl.ds(o, 16)] + ebuf[pl.ds(o, 16)]
            pltpu.sync_copy(xbuf, out_hbm.at[pl.ds(x_off, CHUNK_E)])
            return 0

        lax.fori_loop(0, B, batch_body, 0)
        return 0

    lax.fori_loop(0, N_CHUNK, chunk_body, 0)


def kernel(x, emb_table):
    x_flat = x.reshape(-1)
    emb_flat = emb_table.reshape(-1)
    mesh = plsc.VectorSubcoreMesh(core_axis_name="c", subcore_axis_name="s")
    out = pl.kernel(
        _sc_body,
        mesh=mesh,
        out_type=jax.ShapeDtypeStruct((B * T * D,), jnp.float32),
        scratch_types=[
            pltpu.VMEM((CHUNK_E,), jnp.float32),
            pltpu.VMEM((CHUNK_E,), jnp.float32),
            pltpu.SemaphoreType.DMA,
        ],
    )(x_flat, emb_flat)
    return out.reshape(B, T, D)


# speedup vs baseline: 1.2711x; 1.2711x over previous
import jax
import jax.numpy as jnp
from jax import lax
from jax.experimental import pallas as pl
from jax.experimental.pallas import tpu as pltpu, tpu_sc as plsc

B, T, D = 4, 8192, 1024
NW = 32
T_PER_W = T // NW
CHUNK_T = 16
N_CHUNK = T_PER_W // CHUNK_T
CHUNK_E = CHUNK_T * D
N_STEP = N_CHUNK * B


def _sc_body(x_hbm, emb_hbm, out_hbm, ebuf, ibuf0, ibuf1, obuf0, obuf1,
             lsem0, lsem1, ssem0, ssem1, esem):
    cid = lax.axis_index("c")
    sid = lax.axis_index("s")
    wid = sid * 2 + cid
    t_base = wid * T_PER_W

    ibufs = [ibuf0, ibuf1]
    obufs = [obuf0, obuf1]
    lsems = [lsem0, lsem1]
    ssems = [ssem0, ssem1]

    def x_off(step):
        c = step // B
        b = step % B
        return b * (T * D) + (t_base + c * CHUNK_T) * D

    def e_off(c):
        return (t_base + c * CHUNK_T) * D

    def start_load(step, slot):
        pltpu.async_copy(x_hbm.at[pl.ds(x_off(step), CHUNK_E)], ibufs[slot],
                         lsems[slot])

    pltpu.async_copy(emb_hbm.at[pl.ds(e_off(0), CHUNK_E)], ebuf, esem)
    start_load(0, 0)
    start_load(1, 1)

    def pair_body(p, _):
        for slot in range(2):
            step = p * 2 + slot
            ib = ibufs[slot]
            ob = obufs[slot]

            pltpu.make_async_copy(x_hbm.at[pl.ds(x_off(step), CHUNK_E)], ib,
                                  lsems[slot]).wait()
            @pl.when(step % B == 0)
            def _():
                pltpu.make_async_copy(
                    emb_hbm.at[pl.ds(e_off(step // B), CHUNK_E)], ebuf,
                    esem).wait()

            @pl.when(step >= 2)
            def _():
                pltpu.make_async_copy(ob, out_hbm.at[pl.ds(x_off(step - 2),
                                                           CHUNK_E)],
                                      ssems[slot]).wait()

            @plsc.parallel_loop(0, CHUNK_E, 16, unroll=8)
            def _add(o):
                ob[pl.ds(o, 16)] = ib[pl.ds(o, 16)] + ebuf[pl.ds(o, 16)]

            pltpu.async_copy(ob, out_hbm.at[pl.ds(x_off(step), CHUNK_E)],
                             ssems[slot])

            @pl.when(step + 2 < N_STEP)
            def _():
                start_load(step + 2, slot)

            @pl.when((step % B == B - 1) & (step // B + 1 < N_CHUNK))
            def _():
                pltpu.async_copy(
                    emb_hbm.at[pl.ds(e_off(step // B + 1), CHUNK_E)],
                    ebuf, esem)

        return 0

    lax.fori_loop(0, N_STEP // 2, pair_body, 0)

    pltpu.make_async_copy(obufs[(N_STEP - 2) % 2],
                          out_hbm.at[pl.ds(x_off(N_STEP - 2), CHUNK_E)],
                          ssems[(N_STEP - 2) % 2]).wait()
    pltpu.make_async_copy(obufs[(N_STEP - 1) % 2],
                          out_hbm.at[pl.ds(x_off(N_STEP - 1), CHUNK_E)],
                          ssems[(N_STEP - 1) % 2]).wait()


def kernel(x, emb_table):
    x_flat = x.reshape(-1)
    emb_flat = emb_table.reshape(-1)
    mesh = plsc.VectorSubcoreMesh(core_axis_name="c", subcore_axis_name="s")
    out = pl.kernel(
        _sc_body,
        mesh=mesh,
        out_type=jax.ShapeDtypeStruct((B * T * D,), jnp.float32),
        scratch_types=[
            pltpu.VMEM((CHUNK_E,), jnp.float32),
            pltpu.VMEM((CHUNK_E,), jnp.float32),
            pltpu.VMEM((CHUNK_E,), jnp.float32),
            pltpu.VMEM((CHUNK_E,), jnp.float32),
            pltpu.VMEM((CHUNK_E,), jnp.float32),
            pltpu.SemaphoreType.DMA,
            pltpu.SemaphoreType.DMA,
            pltpu.SemaphoreType.DMA,
            pltpu.SemaphoreType.DMA,
            pltpu.SemaphoreType.DMA,
        ],
    )(x_flat, emb_flat)
    return out.reshape(B, T, D)


# speedup vs baseline: 3.4543x; 2.7177x over previous
import jax
import jax.numpy as jnp
from jax import lax
from jax.experimental import pallas as pl
from jax.experimental.pallas import tpu as pltpu, tpu_sc as plsc

B, T, D = 4, 8192, 1024
NW = 32
T_PER_W = T // NW
CHUNK_T = 16
N_CHUNK = T_PER_W // CHUNK_T
N_STEP = N_CHUNK * B


def _sc_body(x_hbm, emb_hbm, out_hbm, ebuf, ibuf0, ibuf1, obuf0, obuf1,
             lsem0, lsem1, ssem0, ssem1, esem):
    cid = lax.axis_index("c")
    sid = lax.axis_index("s")
    wid = sid * 2 + cid
    t_base = wid * T_PER_W

    ibufs = [ibuf0, ibuf1]
    obufs = [obuf0, obuf1]
    lsems = [lsem0, lsem1]
    ssems = [ssem0, ssem1]

    def t0(step):
        return t_base + (step // B) * CHUNK_T

    def start_load(step, slot):
        pltpu.async_copy(x_hbm.at[step % B, pl.ds(t0(step), CHUNK_T)],
                         ibufs[slot], lsems[slot])

    pltpu.async_copy(emb_hbm.at[pl.ds(t_base, CHUNK_T)], ebuf, esem)
    start_load(0, 0)
    start_load(1, 1)

    def pair_body(p, _):
        for slot in range(2):
            step = p * 2 + slot
            ib = ibufs[slot]
            ob = obufs[slot]

            pltpu.make_async_copy(
                x_hbm.at[step % B, pl.ds(t0(step), CHUNK_T)], ib,
                lsems[slot]).wait()
            @pl.when(step % B == 0)
            def _():
                pltpu.make_async_copy(
                    emb_hbm.at[pl.ds(t0(step), CHUNK_T)], ebuf, esem).wait()

            @pl.when(step >= 2)
            def _():
                pltpu.make_async_copy(
                    ob, out_hbm.at[(step - 2) % B,
                                   pl.ds(t0(step - 2), CHUNK_T)],
                    ssems[slot]).wait()

            for r in range(CHUNK_T):
                @plsc.parallel_loop(0, D, 16, unroll=8)
                def _add(o):
                    ob[r, pl.ds(o, 16)] = (ib[r, pl.ds(o, 16)] +
                                           ebuf[r, pl.ds(o, 16)])

            pltpu.async_copy(ob, out_hbm.at[step % B, pl.ds(t0(step), CHUNK_T)],
                             ssems[slot])

            @pl.when(step + 2 < N_STEP)
            def _():
                start_load(step + 2, slot)

            @pl.when((step % B == B - 1) & (step // B + 1 < N_CHUNK))
            def _():
                pltpu.async_copy(
                    emb_hbm.at[pl.ds(t0(step) + CHUNK_T, CHUNK_T)], ebuf,
                    esem)

        return 0

    lax.fori_loop(0, N_STEP // 2, pair_body, 0)

    pltpu.make_async_copy(obufs[0],
                          out_hbm.at[(N_STEP - 2) % B,
                                     pl.ds(t0(N_STEP - 2), CHUNK_T)],
                          ssems[0]).wait()
    pltpu.make_async_copy(obufs[1],
                          out_hbm.at[(N_STEP - 1) % B,
                                     pl.ds(t0(N_STEP - 1), CHUNK_T)],
                          ssems[1]).wait()


def kernel(x, emb_table):
    mesh = plsc.VectorSubcoreMesh(core_axis_name="c", subcore_axis_name="s")
    return pl.kernel(
        _sc_body,
        mesh=mesh,
        out_type=jax.ShapeDtypeStruct((B, T, D), jnp.float32),
        scratch_types=[
            pltpu.VMEM((CHUNK_T, D), jnp.float32),
            pltpu.VMEM((CHUNK_T, D), jnp.float32),
            pltpu.VMEM((CHUNK_T, D), jnp.float32),
            pltpu.VMEM((CHUNK_T, D), jnp.float32),
            pltpu.VMEM((CHUNK_T, D), jnp.float32),
            pltpu.SemaphoreType.DMA,
            pltpu.SemaphoreType.DMA,
            pltpu.SemaphoreType.DMA,
            pltpu.SemaphoreType.DMA,
            pltpu.SemaphoreType.DMA,
        ],
    )(x, emb_table)


# speedup vs baseline: 3.8667x; 1.1194x over previous
import jax
import jax.numpy as jnp
from jax import lax
from jax.experimental import pallas as pl
from jax.experimental.pallas import tpu as pltpu, tpu_sc as plsc

B, T, D = 4, 8192, 1024
NW = 32
T_PER_W = T // NW
CHUNK_T = 8
N_CHUNK = T_PER_W // CHUNK_T
N_STEP = N_CHUNK * B
DEPTH = 4


def _sc_body(x_hbm, emb_hbm, out_hbm, ebuf0, ebuf1,
             ibuf0, ibuf1, ibuf2, ibuf3, obuf0, obuf1, obuf2, obuf3,
             lsem0, lsem1, lsem2, lsem3, ssem0, ssem1, ssem2, ssem3,
             esem0, esem1):
    cid = lax.axis_index("c")
    sid = lax.axis_index("s")
    wid = sid * 2 + cid
    t_base = wid * T_PER_W

    ebufs = [ebuf0, ebuf1]
    esems = [esem0, esem1]
    ibufs = [ibuf0, ibuf1, ibuf2, ibuf3]
    obufs = [obuf0, obuf1, obuf2, obuf3]
    lsems = [lsem0, lsem1, lsem2, lsem3]
    ssems = [ssem0, ssem1, ssem2, ssem3]

    def c_t0(chunk):
        return t_base + chunk * CHUNK_T

    def start_load(chunk, b):
        pltpu.async_copy(x_hbm.at[b, pl.ds(c_t0(chunk), CHUNK_T)],
                         ibufs[b], lsems[b])

    def start_emb(chunk, eslot):
        pltpu.async_copy(emb_hbm.at[pl.ds(c_t0(chunk), CHUNK_T)],
                         ebufs[eslot], esems[eslot])

    start_emb(0, 0)
    start_emb(1, 1)
    for b in range(B):
        start_load(0, b)

    def pair_body(p2, _):
        for g in range(2):
            chunk = p2 * 2 + g
            eb = ebufs[g]
            for b in range(B):
                ib = ibufs[b]
                ob = obufs[b]

                pltpu.make_async_copy(
                    x_hbm.at[b, pl.ds(c_t0(chunk), CHUNK_T)], ib,
                    lsems[b]).wait()

                if b == 0:
                    pltpu.make_async_copy(
                        emb_hbm.at[pl.ds(c_t0(chunk), CHUNK_T)], eb,
                        esems[g]).wait()

                @pl.when(chunk >= 1)
                def _():
                    pltpu.make_async_copy(
                        ob, out_hbm.at[b, pl.ds(c_t0(chunk - 1), CHUNK_T)],
                        ssems[b]).wait()

                for r in range(CHUNK_T):
                    @plsc.parallel_loop(0, D, 16, unroll=8)
                    def _add(o):
                        ob[r, pl.ds(o, 16)] = (ib[r, pl.ds(o, 16)] +
                                               eb[r, pl.ds(o, 16)])

                pltpu.async_copy(ob,
                                 out_hbm.at[b, pl.ds(c_t0(chunk), CHUNK_T)],
                                 ssems[b])

                @pl.when(chunk + 1 < N_CHUNK)
                def _():
                    start_load(chunk + 1, b)

                if b == B - 1:
                    @pl.when(chunk + 2 < N_CHUNK)
                    def _():
                        start_emb(chunk + 2, g)

        return 0

    lax.fori_loop(0, N_CHUNK // 2, pair_body, 0)

    for b in range(B):
        pltpu.make_async_copy(obufs[b],
                              out_hbm.at[b, pl.ds(c_t0(N_CHUNK - 1), CHUNK_T)],
                              ssems[b]).wait()


def kernel(x, emb_table):
    mesh = plsc.VectorSubcoreMesh(core_axis_name="c", subcore_axis_name="s")
    vm = lambda: pltpu.VMEM((CHUNK_T, D), jnp.float32)
    sem = lambda: pltpu.SemaphoreType.DMA
    return pl.kernel(
        _sc_body,
        mesh=mesh,
        out_type=jax.ShapeDtypeStruct((B, T, D), jnp.float32),
        scratch_types=[vm(), vm(),
                       vm(), vm(), vm(), vm(), vm(), vm(), vm(), vm(),
                       sem(), sem(), sem(), sem(), sem(), sem(), sem(),
                       sem(), sem(), sem()],
    )(x, emb_table)


# speedup vs baseline: 3.9220x; 1.0143x over previous
import jax
import jax.numpy as jnp
from jax import lax
from jax.experimental import pallas as pl
from jax.experimental.pallas import tpu as pltpu, tpu_sc as plsc

B, T, D = 4, 8192, 1024
NW = 32
T_PER_W = T // NW
CHUNK_T = 8
N_CHUNK = T_PER_W // CHUNK_T
N_STEP = N_CHUNK * B


def _sc_body(x_hbm, emb_hbm, out_hbm, ebuf0, ebuf1,
             ibuf0, ibuf1, ibuf2, ibuf3, ibuf4, ibuf5, ibuf6, ibuf7,
             obuf0, obuf1, obuf2, obuf3,
             lsem0, lsem1, lsem2, lsem3, lsem4, lsem5, lsem6, lsem7,
             ssem0, ssem1, ssem2, ssem3, esem0, esem1):
    cid = lax.axis_index("c")
    sid = lax.axis_index("s")
    wid = sid * 2 + cid
    t_base = wid * T_PER_W

    ebufs = [ebuf0, ebuf1]
    esems = [esem0, esem1]
    ibufs = [ibuf0, ibuf1, ibuf2, ibuf3, ibuf4, ibuf5, ibuf6, ibuf7]
    lsems = [lsem0, lsem1, lsem2, lsem3, lsem4, lsem5, lsem6, lsem7]
    obufs = [obuf0, obuf1, obuf2, obuf3]
    ssems = [ssem0, ssem1, ssem2, ssem3]

    def c_t0(chunk):
        return t_base + chunk * CHUNK_T

    def start_load(chunk, b, islot):
        pltpu.async_copy(x_hbm.at[b, pl.ds(c_t0(chunk), CHUNK_T)],
                         ibufs[islot], lsems[islot])

    def start_emb(chunk, eslot):
        pltpu.async_copy(emb_hbm.at[pl.ds(c_t0(chunk), CHUNK_T)],
                         ebufs[eslot], esems[eslot])

    start_emb(0, 0)
    start_emb(1, 1)
    for s in range(8):
        start_load(s // 4, s % 4, s)

    def group_body(p, _):
        for s in range(8):
            chunk = p * 2 + s // 4
            b = s % 4
            ib = ibufs[s]
            ob = obufs[b]
            eb = ebufs[s // 4]

            pltpu.make_async_copy(
                x_hbm.at[b, pl.ds(c_t0(chunk), CHUNK_T)], ib,
                lsems[s]).wait()

            if b == 0:
                pltpu.make_async_copy(
                    emb_hbm.at[pl.ds(c_t0(chunk), CHUNK_T)], eb,
                    esems[s // 4]).wait()

            @pl.when(chunk >= 1)
            def _():
                pltpu.make_async_copy(
                    ob, out_hbm.at[b, pl.ds(c_t0(chunk - 1), CHUNK_T)],
                    ssems[b]).wait()

            for r in range(CHUNK_T):
                @plsc.parallel_loop(0, D, 16, unroll=8)
                def _add(o):
                    ob[r, pl.ds(o, 16)] = (ib[r, pl.ds(o, 16)] +
                                           eb[r, pl.ds(o, 16)])

            pltpu.async_copy(ob, out_hbm.at[b, pl.ds(c_t0(chunk), CHUNK_T)],
                             ssems[b])

            @pl.when(chunk + 2 < N_CHUNK)
            def _():
                start_load(chunk + 2, b, s)

            if b == B - 1:
                @pl.when(chunk + 2 < N_CHUNK)
                def _():
                    start_emb(chunk + 2, s // 4)

        return 0

    lax.fori_loop(0, N_CHUNK // 2, group_body, 0)

    for b in range(B):
        pltpu.make_async_copy(obufs[b],
                              out_hbm.at[b, pl.ds(c_t0(N_CHUNK - 1), CHUNK_T)],
                              ssems[b]).wait()


def kernel(x, emb_table):
    mesh = plsc.VectorSubcoreMesh(core_axis_name="c", subcore_axis_name="s")
    vm = lambda: pltpu.VMEM((CHUNK_T, D), jnp.float32)
    sem = lambda: pltpu.SemaphoreType.DMA
    return pl.kernel(
        _sc_body,
        mesh=mesh,
        out_type=jax.ShapeDtypeStruct((B, T, D), jnp.float32),
        scratch_types=[vm(), vm(),
                       vm(), vm(), vm(), vm(), vm(), vm(), vm(), vm(),
                       vm(), vm(), vm(), vm(),
                       sem(), sem(), sem(), sem(), sem(), sem(), sem(),
                       sem(), sem(), sem(), sem(), sem(), sem(), sem()],
    )(x, emb_table)
